# Initial kernel scaffold; baseline (speedup 1.0000x reference)
import jax
import jax.numpy as jnp
from jax.experimental import pallas as pl


def kernel(x, W, b, acts, mask_idxs):
    raise NotImplementedError("write your pallas kernel here")


# speedup vs baseline: 2.3883x; 2.3883x over previous
import jax
import jax.numpy as jnp
from jax.experimental import pallas as pl
from jax.experimental.pallas import tpu as pltpu


def _fused_body(S, TS, x_ref, w_ref, b_ref, a_ref, m_ref, o_ref):
    i = pl.program_id(0)
    xt = x_ref[...]
    y = jnp.dot(xt.astype(jnp.bfloat16), w_ref[...],
                preferred_element_type=jnp.float32)
    y = y + b_ref[...]
    rows = i * TS + jax.lax.broadcasted_iota(jnp.int32, (TS, 1), 0)
    seq = jax.lax.rem(rows, S)
    hits = seq == m_ref[...]
    onehot = hits.astype(jnp.bfloat16)
    repl = jnp.dot(onehot, a_ref[...], preferred_element_type=jnp.float32)
    hit_any = jnp.any(hits, axis=1, keepdims=True)
    o_ref[...] = jnp.where(hit_any, repl, y)


def kernel(x, W, b, acts, mask_idxs):
    B, S, D = x.shape
    K = mask_idxs.shape[0]
    N = B * S
    TS = 512
    xr = x.reshape(N, D)
    wb = W.astype(jnp.bfloat16)
    ab = acts.astype(jnp.bfloat16)
    br = b.reshape(1, D)
    mr = mask_idxs.reshape(1, K)

    import functools
    body = functools.partial(_fused_body, S, TS)
    out = pl.pallas_call(
        body,
        grid=(N // TS,),
        in_specs=[
            pl.BlockSpec((TS, D), lambda i: (i, 0)),
            pl.BlockSpec((D, D), lambda i: (0, 0)),
            pl.BlockSpec((1, D), lambda i: (0, 0)),
            pl.BlockSpec((K, D), lambda i: (0, 0)),
            pl.BlockSpec((1, K), lambda i: (0, 0)),
        ],
        out_specs=pl.BlockSpec((TS, D), lambda i: (i, 0)),
        out_shape=jax.ShapeDtypeStruct((N, D), jnp.float32),
        compiler_params=pltpu.CompilerParams(
            dimension_semantics=("arbitrary",),
        ),
    )(xr, wb, br, ab, mr)
    return out.reshape(B, S, D)
